# baseline (speedup 1.0000x reference)
import functools

import jax
import jax.numpy as jnp
from jax import lax
from jax.experimental import pallas as pl
from jax.experimental.pallas import tpu as pltpu
from jax.experimental.pallas import tpu_sc as plsc

D = 64
NC, NS = 2, 16
NW = NC * NS
BT = 128
SR = 50
GS = 2
NGW = 26


def _make_gather(s, b):
    assert s == 200 and b == 1024
    mesh = plsc.VectorSubcoreMesh(core_axis_name="c", subcore_axis_name="s")

    @functools.partial(
        pl.kernel,
        mesh=mesh,
        compiler_params=pltpu.CompilerParams(
            use_tc_tiling_on_sc=True, needs_layout_passes=False),
        out_type=jax.ShapeDtypeStruct((s, D, b), jnp.float32),
        scratch_types=[
            pltpu.VMEM((s, BT), jnp.int32),
            pltpu.VMEM((GS, BT, 128), jnp.float32),
            pltpu.VMEM((GS, BT, 128), jnp.float32),
            pltpu.VMEM((GS, D, BT), jnp.float32),
            pltpu.VMEM((GS, D, BT), jnp.float32),
            pltpu.SemaphoreType.DMA,
            pltpu.SemaphoreType.DMA,
            pltpu.SemaphoreType.DMA,
            pltpu.SemaphoreType.DMA,
        ],
    )
    def gather_kernel(words_hbm, table_hbm, out_hbm,
                      idx_v, rows0, rows1, rt0, rt1,
                      gsem0, gsem1, osem0, osem1):
        wid = lax.axis_index("s") * NC + lax.axis_index("c")
        u = wid // 4
        v = wid % 4
        s_base = v * SR
        iota16 = lax.iota(jnp.int32, 16)
        pltpu.sync_copy(words_hbm.at[:, pl.ds(u * BT, BT)], idx_v)

        def s_off(g):
            return jnp.minimum(g * GS, SR - GS)

        def gather_group(g, rows, gsem, start):
            off = s_off(g)
            for ks in range(GS):
                row = s_base + off + ks
                for c in range(BT // 16):
                    reg = idx_v[row, pl.ds(c * 16, 16)]
                    src = table_hbm.at[lax.shift_right_logical(reg, 1)]
                    dst = rows.at[ks, pl.ds(c * 16, 16)]
                    if start:
                        pltpu.async_copy(src, dst, gsem)
                    else:
                        pltpu.make_async_copy(src, dst, gsem).wait()

        def transpose_group(g, rows, rt):
            off = s_off(g)
            for ks in range(GS):
                row = s_base + off + ks
                for c in range(BT // 16):
                    reg = idx_v[row, pl.ds(c * 16, 16)]
                    par = (reg & 1) * 64
                    blk = rows.at[ks, pl.ds(c * 16, 16)]

                    def dbody(d, _, blk=blk, par=par, ks=ks, c=c):
                        vec = plsc.load_gather(blk, [iota16, par + d])
                        rt[ks, d, pl.ds(c * 16, 16)] = vec
                        return 0

                    lax.fori_loop(0, D, dbody, 0)

        def out_copy(g, rt, osem):
            dst = out_hbm.at[pl.ds(s_base + s_off(g), GS), :,
                             pl.ds(u * BT, BT)]
            pltpu.async_copy(rt, dst, osem)
            pltpu.make_async_copy(rt, dst, osem).wait()

        gather_group(0, rows0, gsem0, start=True)
        gather_group(1, rows1, gsem1, start=True)

        def body(k, carry):
            g0 = 2 * k
            gather_group(g0, rows0, gsem0, start=False)
            transpose_group(g0, rows0, rt0)
            gather_group(g0 + 2, rows0, gsem0, start=True)
            out_copy(g0, rt0, osem0)
            gather_group(g0 + 1, rows1, gsem1, start=False)
            transpose_group(g0 + 1, rows1, rt1)
            gather_group(g0 + 3, rows1, gsem1, start=True)
            out_copy(g0 + 1, rt1, osem1)
            return carry

        lax.fori_loop(0, NGW // 2 - 1, body, 0)

        g_last = NGW - 2
        gather_group(g_last, rows0, gsem0, start=False)
        transpose_group(g_last, rows0, rt0)
        out_copy(g_last, rt0, osem0)
        gather_group(g_last + 1, rows1, gsem1, start=False)
        transpose_group(g_last + 1, rows1, rt1)
        out_copy(g_last + 1, rt1, osem1)

    return gather_kernel


def kernel(words, table):
    b, s = words.shape
    packed = table.reshape(table.shape[0] // 2, 2 * table.shape[1])
    out_t = _make_gather(s, b)(words.T, packed)
    return out_t.transpose(2, 0, 1)


# speedup vs baseline: 1.0025x; 1.0025x over previous
import functools

import jax
import jax.numpy as jnp
from jax import lax
from jax.experimental import pallas as pl
from jax.experimental.pallas import tpu as pltpu
from jax.experimental.pallas import tpu_sc as plsc

D = 64
NC, NS = 2, 16
NW = NC * NS
BT = 128
SR = 50
GS = 2
NGW = 26


def _make_gather(s, b):
    assert s == 200 and b == 1024
    mesh = plsc.VectorSubcoreMesh(core_axis_name="c", subcore_axis_name="s")

    @functools.partial(
        pl.kernel,
        mesh=mesh,
        compiler_params=pltpu.CompilerParams(
            use_tc_tiling_on_sc=True, needs_layout_passes=False),
        out_type=jax.ShapeDtypeStruct((s, D, b), jnp.float32),
        scratch_types=[
            pltpu.VMEM((s, BT), jnp.int32),
            pltpu.VMEM((GS, BT, 128), jnp.float32),
            pltpu.VMEM((GS, BT, 128), jnp.float32),
            pltpu.VMEM((GS, D, BT), jnp.float32),
            pltpu.VMEM((GS, D, BT), jnp.float32),
            pltpu.SemaphoreType.DMA,
            pltpu.SemaphoreType.DMA,
            pltpu.SemaphoreType.DMA,
            pltpu.SemaphoreType.DMA,
        ],
    )
    def gather_kernel(words_hbm, table_hbm, out_hbm,
                      idx_v, rows0, rows1, rt0, rt1,
                      gsem0, gsem1, osem0, osem1):
        wid = lax.axis_index("s") * NC + lax.axis_index("c")
        u = wid // 4
        v = wid % 4
        s_base = v * SR
        iota16 = lax.iota(jnp.int32, 16)
        pltpu.sync_copy(words_hbm.at[:, pl.ds(u * BT, BT)], idx_v)

        def s_off(g):
            return jnp.minimum(g * GS, SR - GS)

        def gather_group(g, rows, gsem, start):
            off = s_off(g)
            for ks in range(GS):
                row = s_base + off + ks
                for c in range(BT // 16):
                    reg = idx_v[row, pl.ds(c * 16, 16)]
                    src = table_hbm.at[lax.shift_right_logical(reg, 1)]
                    dst = rows.at[ks, pl.ds(c * 16, 16)]
                    if start:
                        pltpu.async_copy(src, dst, gsem)
                    else:
                        pltpu.make_async_copy(src, dst, gsem).wait()

        def transpose_group(g, rows, rt):
            off = s_off(g)
            for ks in range(GS):
                row = s_base + off + ks
                blk = rows.at[ks]

                def cbody(c, _, blk=blk, row=row, ks=ks):
                    reg = idx_v[row, pl.ds(c * 16, 16)]
                    par = (reg & 1) * 64
                    ridx = c * 16 + iota16
                    for d in range(D):
                        vec = plsc.load_gather(blk, [ridx, par + d])
                        rt[ks, d, pl.ds(c * 16, 16)] = vec
                    return 0

                lax.fori_loop(0, BT // 16, cbody, 0)

        def out_copy(g, rt, osem):
            dst = out_hbm.at[pl.ds(s_base + s_off(g), GS), :,
                             pl.ds(u * BT, BT)]
            pltpu.async_copy(rt, dst, osem)
            pltpu.make_async_copy(rt, dst, osem).wait()

        gather_group(0, rows0, gsem0, start=True)
        gather_group(1, rows1, gsem1, start=True)

        def body(k, carry):
            g0 = 2 * k
            gather_group(g0, rows0, gsem0, start=False)
            transpose_group(g0, rows0, rt0)
            gather_group(g0 + 2, rows0, gsem0, start=True)
            out_copy(g0, rt0, osem0)
            gather_group(g0 + 1, rows1, gsem1, start=False)
            transpose_group(g0 + 1, rows1, rt1)
            gather_group(g0 + 3, rows1, gsem1, start=True)
            out_copy(g0 + 1, rt1, osem1)
            return carry

        lax.fori_loop(0, NGW // 2 - 1, body, 0)

        g_last = NGW - 2
        gather_group(g_last, rows0, gsem0, start=False)
        transpose_group(g_last, rows0, rt0)
        out_copy(g_last, rt0, osem0)
        gather_group(g_last + 1, rows1, gsem1, start=False)
        transpose_group(g_last + 1, rows1, rt1)
        out_copy(g_last + 1, rt1, osem1)

    return gather_kernel


def kernel(words, table):
    b, s = words.shape
    packed = table.reshape(table.shape[0] // 2, 2 * table.shape[1])
    out_t = _make_gather(s, b)(words.T, packed)
    return out_t.transpose(2, 0, 1)


# speedup vs baseline: 1.0150x; 1.0125x over previous
import functools

import jax
import jax.numpy as jnp
from jax import lax
from jax.experimental import pallas as pl
from jax.experimental.pallas import tpu as pltpu
from jax.experimental.pallas import tpu_sc as plsc

D = 64
NC, NS = 2, 16
NW = NC * NS
BT = 128
SR = 50
GS = 1
NGW = 50


def _make_gather(s, b):
    assert s == 200 and b == 1024
    mesh = plsc.VectorSubcoreMesh(core_axis_name="c", subcore_axis_name="s")

    @functools.partial(
        pl.kernel,
        mesh=mesh,
        compiler_params=pltpu.CompilerParams(
            use_tc_tiling_on_sc=True, needs_layout_passes=False),
        out_type=jax.ShapeDtypeStruct((s, D // 8, b // BT, 8, BT), jnp.float32),
        scratch_types=[
            pltpu.VMEM((SR + 6, BT), jnp.int32),
            pltpu.VMEM((GS, BT, 136), jnp.float32),
            pltpu.VMEM((GS, BT, 136), jnp.float32),
            pltpu.VMEM((GS, D // 8, 8, BT), jnp.float32),
            pltpu.VMEM((GS, D // 8, 8, BT), jnp.float32),
            pltpu.SemaphoreType.DMA,
            pltpu.SemaphoreType.DMA,
            pltpu.SemaphoreType.DMA,
            pltpu.SemaphoreType.DMA,
        ],
    )
    def gather_kernel(words_hbm, table_hbm, out_hbm,
                      idx_v, rows0, rows1, rt0, rt1,
                      gsem0, gsem1, osem0, osem1):
        wid = lax.axis_index("s") * NC + lax.axis_index("c")
        u = wid // 4
        v = wid % 4
        s_base = v * SR
        iota16 = lax.iota(jnp.int32, 16)
        abase = (s_base // 8) * 8
        delta = s_base - abase
        pltpu.sync_copy(
            words_hbm.at[pl.ds(abase, SR + 6), pl.ds(u * BT, BT)], idx_v)

        def s_off(g):
            return jnp.minimum(g * GS, SR - GS)

        def gather_group(g, rows, gsem, start):
            off = s_off(g)
            for ks in range(GS):
                row = delta + off + ks
                for c in range(BT // 16):
                    reg = idx_v[row, pl.ds(c * 16, 16)]
                    src = table_hbm.at[lax.shift_right_logical(reg, 1)]
                    dst = rows.at[ks, pl.ds(c * 16, 16), pl.ds(0, 128)]
                    if start:
                        pltpu.async_copy(src, dst, gsem)
                    else:
                        pltpu.make_async_copy(src, dst, gsem).wait()

        def transpose_group(g, rows, rt):
            off = s_off(g)
            for ks in range(GS):
                row = delta + off + ks
                blk = rows.at[ks]

                def cbody(c, _, blk=blk, row=row, ks=ks):
                    reg = idx_v[row, pl.ds(c * 16, 16)]
                    par = (reg & 1) * 64
                    ridx = c * 16 + iota16
                    for d in range(D):
                        vec = plsc.load_gather(blk, [ridx, par + d])
                        rt[ks, d // 8, d % 8, pl.ds(c * 16, 16)] = vec
                    return 0

                lax.fori_loop(0, BT // 16, cbody, 0)

        def out_copy(g, rt, osem):
            dst = out_hbm.at[pl.ds(s_base + s_off(g), GS), :, u]
            pltpu.async_copy(rt, dst, osem)
            pltpu.make_async_copy(rt, dst, osem).wait()

        gather_group(0, rows0, gsem0, start=True)
        gather_group(1, rows1, gsem1, start=True)

        def body(k, carry):
            g0 = 2 * k
            gather_group(g0, rows0, gsem0, start=False)
            transpose_group(g0, rows0, rt0)
            gather_group(g0 + 2, rows0, gsem0, start=True)
            out_copy(g0, rt0, osem0)
            gather_group(g0 + 1, rows1, gsem1, start=False)
            transpose_group(g0 + 1, rows1, rt1)
            gather_group(g0 + 3, rows1, gsem1, start=True)
            out_copy(g0 + 1, rt1, osem1)
            return carry

        lax.fori_loop(0, NGW // 2 - 1, body, 0)

        g_last = NGW - 2
        gather_group(g_last, rows0, gsem0, start=False)
        transpose_group(g_last, rows0, rt0)
        out_copy(g_last, rt0, osem0)
        gather_group(g_last + 1, rows1, gsem1, start=False)
        transpose_group(g_last + 1, rows1, rt1)
        out_copy(g_last + 1, rt1, osem1)

    return gather_kernel


def kernel(words, table):
    b, s = words.shape
    packed = table.reshape(table.shape[0] // 2, 2 * table.shape[1])
    out_t = _make_gather(s, b)(words.T, packed)
    return out_t.transpose(2, 4, 0, 1, 3).reshape(b, s, D)
